# Initial kernel scaffold; baseline (speedup 1.0000x reference)
import jax
import jax.numpy as jnp
from jax.experimental import pallas as pl


def kernel(positions, node_charges, cell, pbc, neighbor_matrix, neighbor_shifts):
    raise NotImplementedError("write your pallas kernel here")


# speedup vs baseline: 29.1249x; 29.1249x over previous
import jax
import jax.numpy as jnp
import numpy as np
from jax import lax
from jax.experimental import pallas as pl
from jax.experimental.pallas import tpu as pltpu
from jax.experimental.pallas import tpu_sc as plsc

N = 10000
NB = 64
NW = 32
ROWS_W = 320
NPAD = NW * ROWS_W
CHUNKS = ROWS_W // 16
L_ALPHA = 0.3
A2 = L_ALPHA * L_ALPHA
CUTOFF = 15.0
KCUT = 0.5
COULOMB = 14.399645351950548
TWOASP = 2.0 * L_ALPHA / float(np.sqrt(np.pi))
SELF_C = L_ALPHA / float(np.sqrt(np.pi))
KPAD = 768
NK = 729
BLK = 256

_NMAX = int(np.ceil(KCUT * 50.0 / (2.0 * np.pi)))
_g = np.arange(-_NMAX, _NMAX + 1)
_KGRID = np.stack(np.meshgrid(_g, _g, _g, indexing="ij"), axis=-1).reshape(-1, 3).astype(np.float32)

_EP = 0.3275911
_EA1 = 0.254829592
_EA2 = -0.284496736
_EA3 = 1.421413741
_EA4 = -1.453152027
_EA5 = 1.061405429

_HI = jax.lax.Precision.HIGHEST


def _bf16_rne(x):
    b = lax.bitcast_convert_type(x, jnp.uint32)
    lsb = (b >> 16) & jnp.uint32(1)
    b = (b + jnp.uint32(0x7FFF) + lsb) & jnp.uint32(0xFFFF0000)
    return lax.bitcast_convert_type(b, jnp.float32)


def _rne16(x):
    b = lax.bitcast_convert_type(x, jnp.int32)
    lsb = (b >> 16) & jnp.int32(1)
    b = (b + jnp.int32(0x7FFF) + lsb) & jnp.int32(-65536)
    return lax.bitcast_convert_type(b, jnp.float32)


def _rsqrt16(x):
    ii = lax.bitcast_convert_type(x, jnp.int32)
    ii = jnp.int32(0x5F3759DF) - (ii >> 1)
    y = lax.bitcast_convert_type(ii, jnp.float32)
    y = y * (1.5 - 0.5 * x * y * y)
    y = y * (1.5 - 0.5 * x * y * y)
    y = y * (1.5 - 0.5 * x * y * y)
    return y


def _pair_quants(dx, dy, dz, qq):
    r2 = dx * dx + dy * dy + dz * dz + 1e-12
    rinv = _rsqrt16(r2)
    r = r2 * rinv
    m = (r > 1e-4) & (r < CUTOFF)
    t = 1.0 / (1.0 + _EP * (L_ALPHA * r))
    poly = t * (_EA1 + t * (_EA2 + t * (_EA3 + t * (_EA4 + t * _EA5))))
    ex = jnp.exp(-A2 * r2)
    pe_raw = qq * poly * ex * rinv
    pe = jnp.where(m, pe_raw, 0.0)
    cp = jnp.where(m, -0.5 * rinv * rinv * (qq * TWOASP * ex + pe_raw), 0.0)
    return pe, cp


def _sc_body(xs_h, ys_h, zs_h, q_h, nb_h, f_out, e_out,
             tabx, taby, tabz, tabq, nb_v, fx, fy, fz, ev):
    w = lax.axis_index("s") * 2 + lax.axis_index("c")
    pltpu.sync_copy(xs_h, tabx)
    pltpu.sync_copy(ys_h, taby)
    pltpu.sync_copy(zs_h, tabz)
    pltpu.sync_copy(q_h, tabq)
    pltpu.sync_copy(nb_h.at[pl.ds(w * (NB * ROWS_W), NB * ROWS_W)], nb_v)

    zero16 = jnp.zeros((16,), jnp.float32)

    def zbody(i, carry):
        fx[pl.ds(i * 16, 16)] = zero16
        fy[pl.ds(i * 16, 16)] = zero16
        fz[pl.ds(i * 16, 16)] = zero16
        return carry

    lax.fori_loop(0, NPAD // 16, zbody, 0)

    base_w = w * ROWS_W

    def chunk(ci, etot):
        rb = ci * 16
        gi = base_w + rb
        xi = tabx[pl.ds(gi, 16)]
        yi = taby[pl.ds(gi, 16)]
        zi = tabz[pl.ds(gi, 16)]
        qi = tabq[pl.ds(gi, 16)]
        xir = _rne16(xi)
        yir = _rne16(yi)
        zir = _rne16(zi)

        def slot(si, ic):
            rfx, rfy, rfz, ea = ic
            jv = nb_v[pl.ds(si * ROWS_W + rb, 16)]
            xj = plsc.load_gather(tabx, [jv])
            yj = plsc.load_gather(taby, [jv])
            zj = plsc.load_gather(tabz, [jv])
            qj = plsc.load_gather(tabq, [jv])
            qq = qi * qj
            pe, _ = _pair_quants(xj - xi, yj - yi, zj - zi, qq)
            dxr = _rne16(xj) - xir
            dyr = _rne16(yj) - yir
            dzr = _rne16(zj) - zir
            _, cpr = _pair_quants(dxr, dyr, dzr, qq)
            wx = cpr * dxr
            wy = cpr * dyr
            wz = cpr * dzr
            plsc.addupdate_scatter(fx, [jv], wx)
            plsc.addupdate_scatter(fy, [jv], wy)
            plsc.addupdate_scatter(fz, [jv], wz)
            return (rfx + wx, rfy + wy, rfz + wz, ea + 0.5 * pe)

        ic0 = (zero16, zero16, zero16, zero16)
        rfx, rfy, rfz, ea = lax.fori_loop(0, NB, slot, ic0)
        fx[pl.ds(gi, 16)] = fx[pl.ds(gi, 16)] - rfx
        fy[pl.ds(gi, 16)] = fy[pl.ds(gi, 16)] - rfy
        fz[pl.ds(gi, 16)] = fz[pl.ds(gi, 16)] - rfz
        return etot + ea - SELF_C * qi * qi

    etot = lax.fori_loop(0, CHUNKS, chunk, zero16)

    ev[...] = etot
    fbase = w * (3 * NPAD)
    pltpu.sync_copy(fx, f_out.at[pl.ds(fbase, NPAD)])
    pltpu.sync_copy(fy, f_out.at[pl.ds(fbase + NPAD, NPAD)])
    pltpu.sync_copy(fz, f_out.at[pl.ds(fbase + 2 * NPAD, NPAD)])
    pltpu.sync_copy(ev, e_out.at[pl.ds(w * 16, 16)])


def _sf_body(pt_ref, kt_ref, s_ref):
    i = pl.program_id(0)
    ph = lax.dot(kt_ref[...], pt_ref[0:4, :])
    c = jnp.cos(ph)
    s = jnp.sin(ph)
    qr = pt_ref[3:4, :]
    sre = jnp.sum(qr * c, axis=1, keepdims=True)
    sim = jnp.sum(qr * s, axis=1, keepdims=True)
    part = jnp.concatenate([sre, sim], axis=1)

    @pl.when(i == 0)
    def _():
        s_ref[...] = jnp.zeros_like(s_ref)

    s_ref[...] += part


def _fc_body(pt_ref, kt_ref, w_ref, fp_ref, o_ref, ckv_ref):
    i = pl.program_id(0)
    qr = pt_ref[3:4, :]
    ph = lax.dot(kt_ref[...], pt_ref[0:4, :])
    c = jnp.cos(ph)
    s = jnp.sin(ph)
    cpt = qr * (c * w_ref[:, 0:1] - s * w_ref[:, 1:2])
    grec = lax.dot_general(kt_ref[...], cpt, (((0,), (0,)), ((), ())))
    fsum = jnp.sum(fp_ref[...], axis=0)
    o_ref[...] = fsum + grec[0:3, :]
    xt4 = pt_ref[4:8, :]
    part = lax.dot_general(cpt, xt4, (((1,), (1,)), ((), ())))

    @pl.when(i == 0)
    def _():
        ckv_ref[...] = jnp.zeros_like(ckv_ref)

    ckv_ref[...] += part


def kernel(positions, node_charges, cell, pbc, neighbor_matrix, neighbor_shifts):
    f32 = jnp.float32
    cell3 = cell[0]
    pos = positions.astype(f32)
    q = node_charges.astype(f32)

    padn = NPAD - N
    xs = jnp.concatenate([pos[:, 0], jnp.zeros((padn,), f32)])
    ys = jnp.concatenate([pos[:, 1], jnp.zeros((padn,), f32)])
    zs = jnp.concatenate([pos[:, 2], jnp.zeros((padn,), f32)])
    qp = jnp.concatenate([q, jnp.zeros((padn,), f32)])
    nbp = jnp.concatenate([neighbor_matrix, jnp.zeros((padn, NB), jnp.int32)], axis=0)
    nbw = nbp.reshape(NW, ROWS_W, NB).transpose(0, 2, 1).reshape(NW, NB * ROWS_W)

    mesh = plsc.VectorSubcoreMesh(core_axis_name="c", subcore_axis_name="s",
                                  num_cores=2, num_subcores=16)
    sc_fn = pl.kernel(
        _sc_body,
        out_type=(
            jax.ShapeDtypeStruct((NW * 3 * NPAD,), f32),
            jax.ShapeDtypeStruct((NW * 16,), f32),
        ),
        mesh=mesh,
        scratch_types=[
            pltpu.VMEM((NPAD,), f32),
            pltpu.VMEM((NPAD,), f32),
            pltpu.VMEM((NPAD,), f32),
            pltpu.VMEM((NPAD,), f32),
            pltpu.VMEM((NB * ROWS_W,), jnp.int32),
            pltpu.VMEM((NPAD,), f32),
            pltpu.VMEM((NPAD,), f32),
            pltpu.VMEM((NPAD,), f32),
            pltpu.VMEM((16,), f32),
        ],
        compiler_params=pltpu.CompilerParams(needs_layout_passes=False),
        name="ewald_real_sc",
    )
    fpart, e_out = sc_fn(xs, ys, zs, qp, nbw.reshape(-1))
    fpart = fpart.reshape(NW, 3, NPAD)

    recip = 2.0 * jnp.pi * jnp.linalg.inv(cell3).T
    kvecs = jnp.asarray(_KGRID) @ recip
    k2 = jnp.sum(kvecs * kvecs, axis=-1)
    valid = (k2 > 1e-9) & (k2 <= KCUT * KCUT)
    vol = jnp.abs(jnp.linalg.det(cell3))
    k2s = jnp.where(valid, k2, 1.0)
    Ak = jnp.where(valid, (4.0 * jnp.pi / k2s) * jnp.exp(-k2 / (4.0 * A2)), 0.0)

    ktab = jnp.zeros((KPAD, 4), f32).at[:NK, :3].set(kvecs)
    xr = _bf16_rne(xs)
    yr = _bf16_rne(ys)
    zr = _bf16_rne(zs)
    PT = jnp.stack([xs, ys, zs, qp, xr, yr, zr, jnp.zeros_like(xs)], axis=0)

    S = pl.pallas_call(
        _sf_body,
        grid=(NPAD // BLK,),
        in_specs=[
            pl.BlockSpec((8, BLK), lambda i: (0, i)),
            pl.BlockSpec((KPAD, 4), lambda i: (0, 0)),
        ],
        out_specs=pl.BlockSpec((KPAD, 2), lambda i: (0, 0)),
        out_shape=jax.ShapeDtypeStruct((KPAD, 2), f32),
    )(PT, ktab)

    Sre = S[:NK, 0]
    Sim = S[:NK, 1]

    w6 = jnp.zeros((KPAD, 8), f32)
    w6 = w6.at[:NK, 0].set(Ak * Sim / vol)
    w6 = w6.at[:NK, 1].set(Ak * Sre / vol)

    gT, ckv = pl.pallas_call(
        _fc_body,
        grid=(NPAD // BLK,),
        in_specs=[
            pl.BlockSpec((8, BLK), lambda i: (0, i)),
            pl.BlockSpec((KPAD, 4), lambda i: (0, 0)),
            pl.BlockSpec((KPAD, 8), lambda i: (0, 0)),
            pl.BlockSpec((NW, 3, BLK), lambda i: (0, 0, i)),
        ],
        out_specs=[
            pl.BlockSpec((3, BLK), lambda i: (0, i)),
            pl.BlockSpec((KPAD, 4), lambda i: (0, 0)),
        ],
        out_shape=[
            jax.ShapeDtypeStruct((3, NPAD), f32),
            jax.ShapeDtypeStruct((KPAD, 4), f32),
        ],
    )(PT, ktab, w6, fpart)

    g_total = gT.T[:N]
    forces = -COULOMB * g_total

    S2 = Sre * Sre + Sim * Sim
    e_recip_sum = jnp.sum(Ak * S2) / (2.0 * vol)
    energies = (jnp.sum(e_out) + e_recip_sum)[None] * COULOMB

    T1 = lax.dot(pos.T, g_total)
    ckv3 = ckv[:NK, 0:3]
    kgrid = jnp.asarray(_KGRID)

    def _cell_tail(eps):
        de = jnp.eye(3, dtype=f32) + eps
        c3d = cell3 @ de
        recip_d = 2.0 * jnp.pi * jnp.linalg.inv(c3d).T
        kv = kgrid @ recip_d
        k2d = jnp.sum(kv * kv, axis=-1)
        validd = (k2d > 1e-9) & (k2d <= KCUT * KCUT)
        vold = jnp.abs(jnp.linalg.det(c3d))
        k2sd = jnp.where(validd, k2d, 1.0)
        Akd = jnp.where(validd, (4.0 * jnp.pi / k2sd) * jnp.exp(-k2d / (4.0 * A2)), 0.0)
        e1 = jnp.sum(Akd * S2) / (2.0 * vold)
        e2 = jnp.sum(ckv3 * kv)
        return e1 + e2

    T2 = jax.grad(_cell_tail)(jnp.zeros((3, 3), f32))
    virial = (T1 + T2) * COULOMB
    stresses = (virial / vol)[None, :, :]
    return (energies, forces, stresses)
